# Initial kernel scaffold; baseline (speedup 1.0000x reference)
import jax
import jax.numpy as jnp
from jax.experimental import pallas as pl


def kernel(x, w0, w1, w2, w3, w4, b0, b1, b2, b3):
    raise NotImplementedError("write your pallas kernel here")


# speedup vs baseline: 1.1891x; 1.1891x over previous
import jax
import jax.numpy as jnp
from jax.experimental import pallas as pl
from jax.experimental.pallas import tpu as pltpu

H = 64
OUT = 3
PACK = 4
OUT_STRIDE = 8
OUT_W = PACK * OUT_STRIDE


def _softplus_unscaled(z):
    return jnp.maximum(z, 0.0) + jnp.log1p(jnp.exp(-jnp.abs(z)))


def _mlp_kernel(x_ref, w0_ref, wh_ref, b_ref, w4_ref, o_ref, *, cm):
    tm = x_ref.shape[0]
    for c in range(tm // cm):
        rows = pl.ds(c * cm, cm)
        h = jnp.dot(x_ref[rows, :], w0_ref[...],
                    preferred_element_type=jnp.float32)
        h = _softplus_unscaled(h + b_ref[0])
        for k in range(3):
            h = jnp.dot(h, wh_ref[k], preferred_element_type=jnp.float32)
            h = _softplus_unscaled(h + b_ref[k + 1])
        o_ref[rows, :] = jnp.dot(h, w4_ref[...],
                                 preferred_element_type=jnp.float32)


def _prep_params(w0, w1, w2, w3, w4, b0, b1, b2, b3, inp_dim, p_in):
    eye = jnp.eye(PACK, dtype=jnp.float32)
    w0s = jnp.zeros((p_in, H), jnp.float32).at[:inp_dim, :].set(w0 * 100.0)
    w0_bd = (eye[:, None, :, None] * w0s[None, :, None, :]
             ).reshape(PACK * p_in, PACK * H)
    wh_bd = jnp.stack(
        [(eye[:, None, :, None] * w[None, :, None, :]).reshape(PACK * H, PACK * H)
         for w in (w1, w2, w3)])
    b_bd = jnp.stack([jnp.tile(b * 100.0, (1, PACK)) for b in (b0, b1, b2, b3)])
    b_bd = b_bd.reshape(4, 1, PACK * H)
    w4s = jnp.zeros((H, OUT_STRIDE), jnp.float32).at[:, :OUT].set(w4 * 0.01)
    w4_bd = (eye[:, None, :, None] * w4s[None, :, None, :]
             ).reshape(PACK * H, OUT_W)
    return w0_bd, wh_bd, b_bd, w4_bd


def kernel(x, w0, w1, w2, w3, w4, b0, b1, b2, b3):
    N, inp_dim = x.shape
    p_in = max(8, -(-inp_dim // 8) * 8)

    np_rows = -(-N // PACK)
    tm = min(4096, max(8, -(-np_rows // 8) * 8))
    n_tiles = -(-np_rows // tm)
    np_pad = n_tiles * tm

    if N == np_pad * PACK and inp_dim == p_in:
        x_packed = x.reshape(np_pad, PACK * p_in)
    else:
        xp = jnp.zeros((np_pad * PACK, p_in), jnp.float32).at[:N, :inp_dim].set(x)
        x_packed = xp.reshape(np_pad, PACK * p_in)

    w0_bd, wh_bd, b_bd, w4_bd = _prep_params(
        w0, w1, w2, w3, w4, b0, b1, b2, b3, inp_dim, p_in)

    def full(a):
        nd = a.ndim
        return pl.BlockSpec(a.shape, lambda i, _nd=nd: (0,) * _nd)

    cm = min(tm, 1024)
    import functools
    out = pl.pallas_call(
        functools.partial(_mlp_kernel, cm=cm),
        out_shape=jax.ShapeDtypeStruct((np_pad, OUT_W), jnp.float32),
        grid=(n_tiles,),
        in_specs=[
            pl.BlockSpec((tm, PACK * p_in), lambda i: (i, 0)),
            full(w0_bd),
            full(wh_bd),
            full(b_bd),
            full(w4_bd),
        ],
        out_specs=pl.BlockSpec((tm, OUT_W), lambda i: (i, 0)),
        compiler_params=pltpu.CompilerParams(
            dimension_semantics=("parallel",)),
    )(x_packed, w0_bd, wh_bd, b_bd, w4_bd)

    return out.reshape(np_pad * PACK, OUT_STRIDE)[:N, :OUT]


# speedup vs baseline: 2.4923x; 2.0960x over previous
import functools

import jax
import jax.numpy as jnp
from jax.experimental import pallas as pl
from jax.experimental.pallas import tpu as pltpu

H = 64
OUT = 3
PACK = 4
OUT_STRIDE = 8
OUT_W = PACK * OUT_STRIDE

_LOG2E = 1.4426950408889634
_LN2 = 0.6931471805599453


def _softplus_unscaled(z):
    t = jnp.exp2(jnp.abs(z) * -_LOG2E)
    return jnp.maximum(z, 0.0) + jnp.log2(1.0 + t) * _LN2


def _mlp_kernel(x_ref, w0_ref, wh_ref, b_ref, w4_ref, o_ref, *, tm, cm):
    for c in range(tm // cm):
        xc = jnp.concatenate(
            [x_ref[pl.ds(s * tm + c * cm, cm), :] for s in range(PACK)],
            axis=1)
        h = jnp.dot(xc, w0_ref[...], preferred_element_type=jnp.float32)
        h = _softplus_unscaled(h + b_ref[0])
        for k in range(3):
            h = jnp.dot(h, wh_ref[k], preferred_element_type=jnp.float32)
            h = _softplus_unscaled(h + b_ref[k + 1])
        res = jnp.dot(h, w4_ref[...], preferred_element_type=jnp.float32)
        for s in range(PACK):
            o_ref[pl.ds(s * tm + c * cm, cm), :] = (
                res[:, s * OUT_STRIDE:(s + 1) * OUT_STRIDE])


def _prep_params(w0, w1, w2, w3, w4, b0, b1, b2, b3, inp_dim, p_in):
    eye = jnp.eye(PACK, dtype=jnp.float32)
    w0s = jnp.zeros((p_in, H), jnp.float32).at[:inp_dim, :].set(w0 * 100.0)
    w0_bd = (eye[:, None, :, None] * w0s[None, :, None, :]
             ).reshape(PACK * p_in, PACK * H)
    wh_bd = jnp.stack(
        [(eye[:, None, :, None] * w[None, :, None, :]).reshape(PACK * H, PACK * H)
         for w in (w1, w2, w3)])
    b_bd = jnp.stack([jnp.tile(b * 100.0, (1, PACK)) for b in (b0, b1, b2, b3)])
    b_bd = b_bd.reshape(4, 1, PACK * H)
    w4s = jnp.zeros((H, OUT_STRIDE), jnp.float32).at[:, :OUT].set(w4 * 0.01)
    w4_bd = (eye[:, None, :, None] * w4s[None, :, None, :]
             ).reshape(PACK * H, OUT_W)
    return w0_bd, wh_bd, b_bd, w4_bd


def kernel(x, w0, w1, w2, w3, w4, b0, b1, b2, b3):
    N, inp_dim = x.shape
    p_in = max(8, -(-inp_dim // 8) * 8)

    np_rows = -(-N // PACK)
    tm = min(4096, max(8, -(-np_rows // 8) * 8))
    n_tiles = -(-np_rows // tm)
    np_pad = n_tiles * tm
    n_pad = np_pad * PACK

    if N == n_pad and inp_dim == p_in:
        x_in = x
    else:
        x_in = jnp.zeros((n_pad, p_in), jnp.float32).at[:N, :inp_dim].set(x)

    w0_bd, wh_bd, b_bd, w4_bd = _prep_params(
        w0, w1, w2, w3, w4, b0, b1, b2, b3, inp_dim, p_in)

    def full(a):
        nd = a.ndim
        return pl.BlockSpec(a.shape, lambda i, _nd=nd: (0,) * _nd)

    cm = min(tm, 1024)
    out = pl.pallas_call(
        functools.partial(_mlp_kernel, tm=tm, cm=cm),
        out_shape=jax.ShapeDtypeStruct((n_pad, OUT_STRIDE), jnp.float32),
        grid=(n_tiles,),
        in_specs=[
            pl.BlockSpec((PACK * tm, p_in), lambda i: (i, 0)),
            full(w0_bd),
            full(wh_bd),
            full(b_bd),
            full(w4_bd),
        ],
        out_specs=pl.BlockSpec((PACK * tm, OUT_STRIDE), lambda i: (i, 0)),
        compiler_params=pltpu.CompilerParams(
            dimension_semantics=("parallel",)),
    )(x_in, w0_bd, wh_bd, b_bd, w4_bd)

    return out[:N, :OUT]


# speedup vs baseline: 2.8247x; 1.1334x over previous
import functools

import jax
import jax.numpy as jnp
from jax.experimental import pallas as pl
from jax.experimental.pallas import tpu as pltpu

H = 64
OUT = 3
PACK = 4
OUT_STRIDE = 8

_C_IN = 100.0 * 1.4426950408889634
_C_OUT = 0.01 * 0.6931471805599453


def _act(z):
    return jnp.maximum(z, 0.0) + jnp.log2(1.0 + jnp.exp2(-jnp.abs(z)))


def _mlp_kernel(x_ref, w0_ref, w1_ref, w2_ref, w3_ref, w4_ref,
                b0_ref, b1_ref, b2_ref, b3_ref, o_ref,
                w0s_ref, whs_ref, w4s_ref, bs_ref, *, tm, cm, p_in):
    @pl.when(pl.program_id(0) == 0)
    def _build():
        w0s_ref[...] = jnp.zeros_like(w0s_ref)
        whs_ref[...] = jnp.zeros_like(whs_ref)
        w4s_ref[...] = jnp.zeros_like(w4s_ref)
        w0v = w0_ref[...] * _C_IN
        whv = (w1_ref[...], w2_ref[...], w3_ref[...])
        w4v = w4_ref[...] * _C_OUT
        for s in range(PACK):
            w0s_ref[p_in * s:p_in * (s + 1), H * s:H * (s + 1)] = w0v
            for k in range(3):
                whs_ref[k, H * s:H * (s + 1), H * s:H * (s + 1)] = whv[k]
            w4s_ref[H * s:H * (s + 1),
                    OUT_STRIDE * s:OUT_STRIDE * s + OUT] = w4v
        for k, b in enumerate((b0_ref, b1_ref, b2_ref, b3_ref)):
            brow = jnp.concatenate([b[...] * _C_IN] * PACK, axis=1)
            bs_ref[k:k + 1, :] = brow

    for c in range(tm // cm):
        xc = jnp.concatenate(
            [x_ref[pl.ds(s * tm + c * cm, cm), :] for s in range(PACK)],
            axis=1)
        h = jnp.dot(xc, w0s_ref[...], preferred_element_type=jnp.float32)
        h = _act(h + bs_ref[0:1, :])
        for k in range(3):
            h = jnp.dot(h, whs_ref[k], preferred_element_type=jnp.float32)
            h = _act(h + bs_ref[k + 1:k + 2, :])
        res = jnp.dot(h, w4s_ref[...], preferred_element_type=jnp.float32)
        for s in range(PACK):
            o_ref[pl.ds(s * tm + c * cm, cm), :] = (
                res[:, OUT_STRIDE * s:OUT_STRIDE * s + OUT])


def kernel(x, w0, w1, w2, w3, w4, b0, b1, b2, b3):
    N, inp_dim = x.shape
    p_in = max(8, -(-inp_dim // 8) * 8)

    np_rows = -(-N // PACK)
    tm = min(4096, max(8, -(-np_rows // 8) * 8))
    n_tiles = -(-np_rows // tm)
    np_pad = n_tiles * tm
    n_pad = np_pad * PACK

    if N == n_pad and inp_dim == p_in:
        x_in = x
    else:
        x_in = jnp.zeros((n_pad, p_in), jnp.float32).at[:N, :inp_dim].set(x)

    def full(a):
        nd = a.ndim
        return pl.BlockSpec(a.shape, lambda i, _nd=nd: (0,) * _nd)

    cm = min(tm, 1024)
    out = pl.pallas_call(
        functools.partial(_mlp_kernel, tm=tm, cm=cm, p_in=p_in),
        out_shape=jax.ShapeDtypeStruct((n_pad, OUT), jnp.float32),
        grid=(n_tiles,),
        in_specs=[
            pl.BlockSpec((PACK * tm, p_in), lambda i: (i, 0)),
            full(w0), full(w1), full(w2), full(w3), full(w4),
            full(b0), full(b1), full(b2), full(b3),
        ],
        out_specs=pl.BlockSpec((PACK * tm, OUT), lambda i: (i, 0)),
        scratch_shapes=[
            pltpu.VMEM((PACK * p_in, PACK * H), jnp.float32),
            pltpu.VMEM((3, PACK * H, PACK * H), jnp.float32),
            pltpu.VMEM((PACK * H, PACK * OUT_STRIDE), jnp.float32),
            pltpu.VMEM((8, PACK * H), jnp.float32),
        ],
        compiler_params=pltpu.CompilerParams(
            dimension_semantics=("parallel",)),
    )(x_in, w0, w1, w2, w3, w4, b0, b1, b2, b3)

    return out if N == n_pad else out[:N, :]


# speedup vs baseline: 3.5737x; 1.2651x over previous
import functools

import jax
import jax.numpy as jnp
from jax.experimental import pallas as pl
from jax.experimental.pallas import tpu as pltpu

H = 64
OUT = 3
PACK = 4
OUT_STRIDE = 8

_C_IN = 100.0 * 1.4426950408889634
_C_OUT = 0.01 * 0.6931471805599453


def _act(z):
    return jnp.maximum(z, 0.0) + jnp.log2(1.0 + jnp.exp2(-jnp.abs(z)))


def _mlp_kernel(x_ref, w0_ref, w1_ref, w2_ref, w3_ref, w4_ref,
                b0_ref, b1_ref, b2_ref, b3_ref, o_ref,
                w0s_ref, whs_ref, w4s_ref, bs_ref, *, tm, cm, p_in):
    @pl.when(pl.program_id(0) == 0)
    def _build():
        w0s_ref[...] = jnp.zeros_like(w0s_ref)
        whs_ref[...] = jnp.zeros_like(whs_ref)
        w4s_ref[...] = jnp.zeros_like(w4s_ref)
        w0v = w0_ref[...] * _C_IN
        whv = (w1_ref[...], w2_ref[...], w3_ref[...])
        w4v = w4_ref[...] * _C_OUT
        for s in range(PACK):
            w0s_ref[p_in * s:p_in * (s + 1), H * s:H * (s + 1)] = w0v
            for k in range(3):
                whs_ref[k, H * s:H * (s + 1), H * s:H * (s + 1)] = whv[k]
            w4s_ref[H * s:H * (s + 1),
                    OUT_STRIDE * s:OUT_STRIDE * s + OUT] = w4v
        for k, b in enumerate((b0_ref, b1_ref, b2_ref, b3_ref)):
            brow = jnp.concatenate([b[...] * _C_IN] * PACK, axis=1)
            bs_ref[k:k + 1, :] = brow

    for c in range(tm // cm):
        xc = jnp.concatenate(
            [x_ref[pl.ds(s * tm + c * cm, cm), :] for s in range(PACK)],
            axis=1)
        h = jnp.dot(xc, w0s_ref[...], preferred_element_type=jnp.float32)
        h = _act(h + bs_ref[0:1, :])
        for k in range(3):
            h = jnp.dot(h, whs_ref[k], preferred_element_type=jnp.float32)
            h = _act(h + bs_ref[k + 1:k + 2, :])
        res = jnp.dot(h, w4s_ref[...], preferred_element_type=jnp.float32)
        o_ref[pl.ds(c * cm, cm), :] = res


def kernel(x, w0, w1, w2, w3, w4, b0, b1, b2, b3):
    N, inp_dim = x.shape
    p_in = max(8, -(-inp_dim // 8) * 8)

    np_rows = -(-N // PACK)
    tm = min(4096, max(8, -(-np_rows // 8) * 8))
    n_tiles = -(-np_rows // tm)
    np_pad = n_tiles * tm
    n_pad = np_pad * PACK

    if N == n_pad and inp_dim == p_in:
        x_in = x
    else:
        x_in = jnp.zeros((n_pad, p_in), jnp.float32).at[:N, :inp_dim].set(x)

    def full(a):
        nd = a.ndim
        return pl.BlockSpec(a.shape, lambda i, _nd=nd: (0,) * _nd)

    cm = min(tm, 1024)
    out = pl.pallas_call(
        functools.partial(_mlp_kernel, tm=tm, cm=cm, p_in=p_in),
        out_shape=jax.ShapeDtypeStruct((np_pad, PACK * OUT_STRIDE), jnp.float32),
        grid=(n_tiles,),
        in_specs=[
            pl.BlockSpec((PACK * tm, p_in), lambda i: (i, 0)),
            full(w0), full(w1), full(w2), full(w3), full(w4),
            full(b0), full(b1), full(b2), full(b3),
        ],
        out_specs=pl.BlockSpec((tm, PACK * OUT_STRIDE), lambda i: (i, 0)),
        scratch_shapes=[
            pltpu.VMEM((PACK * p_in, PACK * H), jnp.float32),
            pltpu.VMEM((3, PACK * H, PACK * H), jnp.float32),
            pltpu.VMEM((PACK * H, PACK * OUT_STRIDE), jnp.float32),
            pltpu.VMEM((8, PACK * H), jnp.float32),
        ],
        compiler_params=pltpu.CompilerParams(
            dimension_semantics=("parallel",)),
    )(x_in, w0, w1, w2, w3, w4, b0, b1, b2, b3)

    y = out.reshape(n_tiles, tm, PACK, OUT_STRIDE).transpose(0, 2, 1, 3)
    return y.reshape(n_pad, OUT_STRIDE)[:N, :OUT]


# speedup vs baseline: 4.0857x; 1.1433x over previous
import functools

import jax
import jax.numpy as jnp
from jax.experimental import pallas as pl
from jax.experimental.pallas import tpu as pltpu

H = 64
OUT = 3
PACK = 4
OUT_STRIDE = 4
ROWGROUP = 8

_C_IN = 100.0 * 1.4426950408889634
_C_OUT = 0.01 * 0.6931471805599453


def _act(z):
    return jnp.maximum(z, 0.0) + jnp.log2(1.0 + jnp.exp2(-jnp.abs(z)))


def _mlp_kernel(x_ref, w0_ref, w1_ref, w2_ref, w3_ref, w4_ref,
                b0_ref, b1_ref, b2_ref, b3_ref, o_ref,
                w0s_ref, whs_ref, w4s_ref, bs_ref, *, tm, cm, p_in):
    @pl.when(pl.program_id(0) == 0)
    def _build():
        w0s_ref[...] = jnp.zeros_like(w0s_ref)
        whs_ref[...] = jnp.zeros_like(whs_ref)
        w4s_ref[...] = jnp.zeros_like(w4s_ref)
        w0v = w0_ref[...] * _C_IN
        whv = (w1_ref[...], w2_ref[...], w3_ref[...])
        w4v = w4_ref[...] * _C_OUT
        for s in range(PACK):
            w0s_ref[p_in * s:p_in * (s + 1), H * s:H * (s + 1)] = w0v
            for k in range(3):
                whs_ref[k, H * s:H * (s + 1), H * s:H * (s + 1)] = whv[k]
            w4s_ref[H * s:H * (s + 1),
                    OUT_STRIDE * s:OUT_STRIDE * s + OUT] = w4v
        for k, b in enumerate((b0_ref, b1_ref, b2_ref, b3_ref)):
            brow = jnp.concatenate([b[...] * _C_IN] * PACK, axis=1)
            bs_ref[k:k + 1, :] = brow

    for c in range(tm // cm):
        xc = jnp.concatenate(
            [x_ref[pl.ds(s * tm + c * cm, cm), :] for s in range(PACK)],
            axis=1)
        h = jnp.dot(xc, w0s_ref[...], preferred_element_type=jnp.float32)
        h = _act(h + bs_ref[0:1, :])
        for k in range(3):
            h = jnp.dot(h, whs_ref[k], preferred_element_type=jnp.float32)
            h = _act(h + bs_ref[k + 1:k + 2, :])
        res = jnp.dot(h, w4s_ref[...], preferred_element_type=jnp.float32)
        o_ref[:, pl.ds(c * cm, cm)] = res.T


def kernel(x, w0, w1, w2, w3, w4, b0, b1, b2, b3):
    N, inp_dim = x.shape
    p_in = max(8, -(-inp_dim // 8) * 8)

    np_rows = -(-N // PACK)
    tm = min(4096, max(8, -(-np_rows // 8) * 8))
    n_tiles = -(-np_rows // tm)
    np_pad = n_tiles * tm
    n_pad = np_pad * PACK

    if N == n_pad and inp_dim == p_in:
        x_in = x
    else:
        x_in = jnp.zeros((n_pad, p_in), jnp.float32).at[:N, :inp_dim].set(x)

    def full(a):
        nd = a.ndim
        return pl.BlockSpec(a.shape, lambda i, _nd=nd: (0,) * _nd)

    cm = min(tm, 1024)
    out = pl.pallas_call(
        functools.partial(_mlp_kernel, tm=tm, cm=cm, p_in=p_in),
        out_shape=jax.ShapeDtypeStruct(
            (PACK * OUT_STRIDE, np_pad), jnp.float32),
        grid=(n_tiles,),
        in_specs=[
            pl.BlockSpec((PACK * tm, p_in), lambda i: (i, 0)),
            full(w0), full(w1), full(w2), full(w3), full(w4),
            full(b0), full(b1), full(b2), full(b3),
        ],
        out_specs=pl.BlockSpec((PACK * OUT_STRIDE, tm), lambda i: (0, i)),
        scratch_shapes=[
            pltpu.VMEM((PACK * p_in, PACK * H), jnp.float32),
            pltpu.VMEM((3, PACK * H, PACK * H), jnp.float32),
            pltpu.VMEM((PACK * H, PACK * OUT_STRIDE), jnp.float32),
            pltpu.VMEM((8, PACK * H), jnp.float32),
        ],
        compiler_params=pltpu.CompilerParams(
            dimension_semantics=("parallel",)),
    )(x_in, w0, w1, w2, w3, w4, b0, b1, b2, b3)

    y = out.reshape(PACK, OUT_STRIDE, n_tiles, tm)
    y = y.transpose(2, 0, 3, 1)
    return y.reshape(n_pad, OUT_STRIDE)[:N, :OUT]
